# Initial kernel scaffold; baseline (speedup 1.0000x reference)
import jax
import jax.numpy as jnp
from jax.experimental import pallas as pl


def kernel(A, h, W_l0_c1, W_l0_c2, W_l1_c1, gcn_W, gcn_b):
    raise NotImplementedError("write your pallas kernel here")


# speedup vs baseline: 4.3262x; 4.3262x over previous
import functools

import jax
import jax.numpy as jnp
from jax.experimental import pallas as pl
from jax.experimental.pallas import tpu as pltpu

_BM = 512
_BK = 512


def _expm1_poly(x):
    return x * (1.0 + x * (0.5 + x * ((1.0 / 6.0) + x * ((1.0 / 24.0)
                + x * ((1.0 / 120.0) + x * (1.0 / 720.0))))))


def _hw_body(h_ref, w_ref, o_ref, *, C):
    for c in range(C):
        o_ref[c] = jnp.dot(h_ref[...], w_ref[c],
                           preferred_element_type=jnp.float32)


def _pass0_body(A_ref, W1, W2, W3, hwa_ref, b_ref,
                e1_ref, e2_ref, out0_ref, P3_ref,
                acc1, acc3, cs, *, C, T, DO):
    k = pl.program_id(1)
    nk = pl.num_programs(1)

    @pl.when(k == 0)
    def _init():
        acc1[...] = jnp.zeros_like(acc1)
        acc3[...] = jnp.zeros_like(acc3)
        cs[...] = jnp.zeros_like(cs)

    a = [A_ref[t] for t in range(T)]
    for c in range(C):
        b1 = W1[c, 0] * a[0]
        b2 = W2[c, 0] * a[0]
        b3 = W3[c, 0] * a[0]
        for t in range(1, T):
            b1 += W1[c, t] * a[t]
            b2 += W2[c, t] * a[t]
            b3 += W3[c, t] * a[t]
        e1 = _expm1_poly(b1).astype(jnp.bfloat16)
        e2 = _expm1_poly(b2).astype(jnp.bfloat16)
        e1_ref[c] = e1
        e2_ref[c] = e2
        hwa = hwa_ref[c]
        acc1[c] += jnp.dot(e1, hwa, preferred_element_type=jnp.float32)
        acc3[c] += jnp.dot(b3.astype(jnp.bfloat16), hwa,
                           preferred_element_type=jnp.float32)
        cs[c] += jnp.sum(hwa.astype(jnp.float32), axis=0, keepdims=True)

    @pl.when(k == nk - 1)
    def _fin():
        for c in range(C):
            full1 = acc1[c] + cs[c]
            rs1 = full1[:, DO:DO + 1]
            out0_ref[c] = full1[:, 0:DO] / rs1 + b_ref[c]
            P3_ref[c] = acc3[c]


def _pass2_body(e_ref, x_ref, o_ref, acc, cs, *, C, XW):
    k = pl.program_id(1)
    nk = pl.num_programs(1)

    @pl.when(k == 0)
    def _init():
        acc[...] = jnp.zeros_like(acc)
        cs[...] = jnp.zeros_like(cs)

    for c in range(C):
        x = x_ref[c]
        acc[c] += jnp.dot(e_ref[c], x, preferred_element_type=jnp.float32)
        cs[c] += jnp.sum(x.astype(jnp.float32), axis=0, keepdims=True)

    @pl.when(k == nk - 1)
    def _fin():
        for c in range(C):
            full = acc[c] + cs[c]
            rs = full[:, XW - 1:XW]
            o_ref[c] = full / rs


def _pass3_body(e_ref, z_ref, b_ref, o_ref, acc, cs, *, C, DO):
    k = pl.program_id(1)
    nk = pl.num_programs(1)

    @pl.when(k == 0)
    def _init():
        acc[...] = jnp.zeros_like(acc)
        cs[...] = jnp.zeros_like(cs)

    for c in range(C):
        z = z_ref[c]
        acc[c] += jnp.dot(e_ref[c], z, preferred_element_type=jnp.float32)
        cs[c] += jnp.sum(z.astype(jnp.float32), axis=0, keepdims=True)

    @pl.when(k == nk - 1)
    def _fin():
        for c in range(C):
            full = acc[c] + cs[c]
            rs1 = full[:, 2 * DO + 1:2 * DO + 2]
            r = full / rs1
            s2 = r[:, 2 * DO:2 * DO + 1]
            s2 = jnp.where(s2 == 0.0, 1.0, s2)
            o_ref[c, 0] = r[:, 0:DO] + b_ref[c]
            o_ref[c, 1] = r[:, DO:2 * DO] / s2 + b_ref[c]


def kernel(A, h, W_l0_c1, W_l0_c2, W_l1_c1, gcn_W, gcn_b):
    T, N, _ = A.shape
    C, D, DO = gcn_W.shape
    BM, BK = _BM, _BK
    HWW = DO + 1
    XW = 2 * DO + 2
    grid = (N // BM, N // BK)
    params = pltpu.CompilerParams(
        dimension_semantics=("parallel", "arbitrary"))

    hw = pl.pallas_call(
        functools.partial(_hw_body, C=C),
        out_shape=jax.ShapeDtypeStruct((C, N, DO), jnp.float32),
    )(h, gcn_W)

    ones = jnp.ones((C, N, 1), jnp.float32)
    hwa = jnp.concatenate([hw, ones], axis=-1).astype(jnp.bfloat16)
    b2d = gcn_b.reshape(C, 1, DO)

    e1, e2, out0, P3 = pl.pallas_call(
        functools.partial(_pass0_body, C=C, T=T, DO=DO),
        grid=grid,
        in_specs=[
            pl.BlockSpec((T, BM, BK), lambda i, k: (0, i, k)),
            pl.BlockSpec(memory_space=pltpu.SMEM),
            pl.BlockSpec(memory_space=pltpu.SMEM),
            pl.BlockSpec(memory_space=pltpu.SMEM),
            pl.BlockSpec((C, BK, HWW), lambda i, k: (0, k, 0)),
            pl.BlockSpec((C, 1, DO), lambda i, k: (0, 0, 0)),
        ],
        out_specs=[
            pl.BlockSpec((C, BM, BK), lambda i, k: (0, i, k)),
            pl.BlockSpec((C, BM, BK), lambda i, k: (0, i, k)),
            pl.BlockSpec((C, BM, DO), lambda i, k: (0, i, 0)),
            pl.BlockSpec((C, BM, HWW), lambda i, k: (0, i, 0)),
        ],
        out_shape=[
            jax.ShapeDtypeStruct((C, N, N), jnp.bfloat16),
            jax.ShapeDtypeStruct((C, N, N), jnp.bfloat16),
            jax.ShapeDtypeStruct((C, N, DO), jnp.float32),
            jax.ShapeDtypeStruct((C, N, HWW), jnp.float32),
        ],
        scratch_shapes=[
            pltpu.VMEM((C, BM, HWW), jnp.float32),
            pltpu.VMEM((C, BM, HWW), jnp.float32),
            pltpu.VMEM((C, 1, HWW), jnp.float32),
        ],
        compiler_params=params,
    )(A, W_l0_c1, W_l0_c2, W_l1_c1, hwa, b2d)

    X = jnp.concatenate([hw, P3, ones], axis=-1).astype(jnp.bfloat16)

    RBX = pl.pallas_call(
        functools.partial(_pass2_body, C=C, XW=XW),
        grid=grid,
        in_specs=[
            pl.BlockSpec((C, BM, BK), lambda i, k: (0, i, k)),
            pl.BlockSpec((C, BK, XW), lambda i, k: (0, k, 0)),
        ],
        out_specs=pl.BlockSpec((C, BM, XW), lambda i, k: (0, i, 0)),
        out_shape=jax.ShapeDtypeStruct((C, N, XW), jnp.float32),
        scratch_shapes=[
            pltpu.VMEM((C, BM, XW), jnp.float32),
            pltpu.VMEM((C, 1, XW), jnp.float32),
        ],
        compiler_params=params,
    )(e2, X)

    Z = RBX.astype(jnp.bfloat16)

    out12 = pl.pallas_call(
        functools.partial(_pass3_body, C=C, DO=DO),
        grid=grid,
        in_specs=[
            pl.BlockSpec((C, BM, BK), lambda i, k: (0, i, k)),
            pl.BlockSpec((C, BK, XW), lambda i, k: (0, k, 0)),
            pl.BlockSpec((C, 1, DO), lambda i, k: (0, 0, 0)),
        ],
        out_specs=pl.BlockSpec((C, 2, BM, DO), lambda i, k: (0, 0, i, 0)),
        out_shape=jax.ShapeDtypeStruct((C, 2, N, DO), jnp.float32),
        scratch_shapes=[
            pltpu.VMEM((C, BM, XW), jnp.float32),
            pltpu.VMEM((C, 1, XW), jnp.float32),
        ],
        compiler_params=params,
    )(e1, Z, b2d)

    return jnp.concatenate([out0[:, None], out12], axis=1)


# speedup vs baseline: 7.6615x; 1.7710x over previous
import functools

import jax
import jax.numpy as jnp
from jax.experimental import pallas as pl
from jax.experimental.pallas import tpu as pltpu

_BM0 = 128
_BM2 = 256


def _expm1_poly(x):
    return x * (1.0 + x * (0.5 + x * ((1.0 / 6.0) + x * (1.0 / 24.0))))


def _body(A_ref, W1, W2, W3, h_ref, w_ref, b_ref, o_ref,
          e1_s, e2_s, X_s, Z_s, out0_s, hwa_s,
          *, C, T, DO, BM0, NB0, BM2, NB2):
    g = pl.program_id(0)

    @pl.when(g == 0)
    def _init():
        n = h_ref.shape[0]
        ones = jnp.ones((n, 1), jnp.float32)
        for c in range(C):
            hw = jnp.dot(h_ref[...], w_ref[c],
                         preferred_element_type=jnp.float32)
            hwa_s[c] = jnp.concatenate([hw, ones], axis=1).astype(jnp.bfloat16)

    @pl.when(g < NB0)
    def _phase0():
        rows = pl.ds(g * BM0, BM0)
        ab = [A_ref[t].astype(jnp.bfloat16) for t in range(T)]
        for c in range(C):
            w1 = [W1[c, t].astype(jnp.bfloat16) for t in range(T)]
            w2 = [W2[c, t].astype(jnp.bfloat16) for t in range(T)]
            b1 = w1[0] * ab[0]
            b2 = w2[0] * ab[0]
            b3 = W3[c, 0] * A_ref[0]
            for t in range(1, T):
                b1 += w1[t] * ab[t]
                b2 += w2[t] * ab[t]
                b3 += W3[c, t] * A_ref[t]
            e1 = _expm1_poly(b1)
            e2 = _expm1_poly(b2)
            e1_s[c, rows, :] = e1
            e2_s[c, rows, :] = e2
            hwa = hwa_s[c]
            b3h = b3.astype(jnp.bfloat16)
            b3l = (b3 - b3h.astype(jnp.float32)).astype(jnp.bfloat16)
            p3 = (jnp.dot(b3h, hwa, preferred_element_type=jnp.float32)
                  + jnp.dot(b3l, hwa, preferred_element_type=jnp.float32))
            rs3 = p3[:, DO:DO + 1]
            rs3h = rs3.astype(jnp.bfloat16)
            rs3l = (rs3 - rs3h.astype(jnp.float32)).astype(jnp.bfloat16)
            X_s[c, rows, :] = jnp.concatenate(
                [hwa_s[c, rows, :], p3[:, 0:DO].astype(jnp.bfloat16),
                 rs3h, rs3l], axis=1)
            cs = jnp.sum(hwa.astype(jnp.float32), axis=0, keepdims=True)
            full1 = jnp.dot(e1, hwa, preferred_element_type=jnp.float32) + cs
            rs1 = full1[:, DO:DO + 1]
            out0_s[c, rows, :] = full1[:, 0:DO] / rs1 + b_ref[c]

    @pl.when((g >= NB0) & (g < NB0 + NB2))
    def _phase1():
        rows = pl.ds((g - NB0) * BM2, BM2)
        for c in range(C):
            x = X_s[c]
            cs = jnp.sum(x.astype(jnp.float32), axis=0, keepdims=True)
            full = jnp.dot(e2_s[c, rows, :], x,
                           preferred_element_type=jnp.float32) + cs
            rs = full[:, DO:DO + 1]
            v = (full[:, 2 * DO + 1:2 * DO + 2]
                 + full[:, 2 * DO + 2:2 * DO + 3]) / rs
            vh = v.astype(jnp.bfloat16)
            vl = (v - vh.astype(jnp.float32)).astype(jnp.bfloat16)
            zmain = (full[:, 0:2 * DO + 1] / rs).astype(jnp.bfloat16)
            Z_s[c, rows, :] = jnp.concatenate([zmain, vh, vl], axis=1)

    @pl.when(g >= NB0 + NB2)
    def _phase2():
        rows = pl.ds((g - NB0 - NB2) * BM2, BM2)
        for c in range(C):
            z = Z_s[c]
            cs = jnp.sum(z.astype(jnp.float32), axis=0, keepdims=True)
            full = jnp.dot(e1_s[c, rows, :], z,
                           preferred_element_type=jnp.float32) + cs
            rs1 = full[:, DO:DO + 1]
            s2 = (full[:, 2 * DO + 1:2 * DO + 2]
                  + full[:, 2 * DO + 2:2 * DO + 3]) / rs1
            s2 = jnp.where(s2 == 0.0, 1.0, s2)
            o_ref[c, 0] = out0_s[c, rows, :]
            o_ref[c, 1] = full[:, 0:DO] / rs1 + b_ref[c]
            o_ref[c, 2] = full[:, DO + 1:2 * DO + 1] / (rs1 * s2) + b_ref[c]


def kernel(A, h, W_l0_c1, W_l0_c2, W_l1_c1, gcn_W, gcn_b):
    T, N, _ = A.shape
    C, D, DO = gcn_W.shape
    BM0, BM2 = _BM0, _BM2
    NB0, NB2 = N // BM0, N // BM2
    HWW = DO + 1
    XW = 2 * DO + 3
    b2d = gcn_b.reshape(C, 1, DO)

    return pl.pallas_call(
        functools.partial(_body, C=C, T=T, DO=DO,
                          BM0=BM0, NB0=NB0, BM2=BM2, NB2=NB2),
        grid=(NB0 + 2 * NB2,),
        in_specs=[
            pl.BlockSpec((T, BM0, N),
                         lambda g: (0, jnp.minimum(g, NB0 - 1), 0)),
            pl.BlockSpec(memory_space=pltpu.SMEM),
            pl.BlockSpec(memory_space=pltpu.SMEM),
            pl.BlockSpec(memory_space=pltpu.SMEM),
            pl.BlockSpec((N, D), lambda g: (0, 0)),
            pl.BlockSpec((C, D, DO), lambda g: (0, 0, 0)),
            pl.BlockSpec((C, 1, DO), lambda g: (0, 0, 0)),
        ],
        out_specs=pl.BlockSpec(
            (C, 3, BM2, DO),
            lambda g: (0, 0, jnp.maximum(g - (NB0 + NB2), 0), 0)),
        out_shape=jax.ShapeDtypeStruct((C, 3, N, DO), jnp.float32),
        scratch_shapes=[
            pltpu.VMEM((C, N, N), jnp.bfloat16),
            pltpu.VMEM((C, N, N), jnp.bfloat16),
            pltpu.VMEM((C, N, XW), jnp.bfloat16),
            pltpu.VMEM((C, N, XW), jnp.bfloat16),
            pltpu.VMEM((C, N, DO), jnp.float32),
            pltpu.VMEM((C, N, HWW), jnp.bfloat16),
        ],
        compiler_params=pltpu.CompilerParams(
            dimension_semantics=("arbitrary",)),
    )(A, W_l0_c1, W_l0_c2, W_l1_c1, h, gcn_W, b2d)


# speedup vs baseline: 8.3046x; 1.0839x over previous
import functools

import jax
import jax.numpy as jnp
from jax.experimental import pallas as pl
from jax.experimental.pallas import tpu as pltpu

_BM0 = 256
_BM2 = 512


def _expm1_poly(x):
    return x * (1.0 + x * (0.5 + x * ((1.0 / 6.0) + x * (1.0 / 24.0))))


def _body(A_ref, W1, W2, W3, h_ref, w_ref, b_ref, o_ref,
          e1_s, e2_s, X_s, Z_s, hwa_s,
          *, C, T, DO, BM0, NB0, BM2, NB2):
    g = pl.program_id(0)

    @pl.when(g == 0)
    def _init():
        n = h_ref.shape[0]
        ones = jnp.ones((n, 1), jnp.float32)
        for c in range(C):
            hw = jnp.dot(h_ref[...], w_ref[c],
                         preferred_element_type=jnp.float32)
            hwa_s[c] = jnp.concatenate([hw, ones], axis=1).astype(jnp.bfloat16)

    @pl.when(g < NB0)
    def _phase0():
        rows = pl.ds(g * BM0, BM0)
        ab = [A_ref[t].astype(jnp.bfloat16) for t in range(T)]
        for c in range(C):
            w1 = [W1[c, t].astype(jnp.bfloat16) for t in range(T)]
            w2 = [W2[c, t].astype(jnp.bfloat16) for t in range(T)]
            b1 = w1[0] * ab[0]
            b2 = w2[0] * ab[0]
            b3 = W3[c, 0] * A_ref[0]
            for t in range(1, T):
                b1 += w1[t] * ab[t]
                b2 += w2[t] * ab[t]
                b3 += W3[c, t] * A_ref[t]
            e1 = _expm1_poly(b1)
            e2 = _expm1_poly(b2)
            e1_s[c, rows, :] = e1
            e2_s[c, rows, :] = e2
            hwa = hwa_s[c]
            b3h = b3.astype(jnp.bfloat16)
            b3l = (b3 - b3h.astype(jnp.float32)).astype(jnp.bfloat16)
            p3 = (jnp.dot(b3h, hwa, preferred_element_type=jnp.float32)
                  + jnp.dot(b3l, hwa, preferred_element_type=jnp.float32))
            rs3 = p3[:, DO:DO + 1]
            rs3h = rs3.astype(jnp.bfloat16)
            rs3l = (rs3 - rs3h.astype(jnp.float32)).astype(jnp.bfloat16)
            X_s[c, rows, :] = jnp.concatenate(
                [hwa_s[c, rows, :], p3[:, 0:DO].astype(jnp.bfloat16),
                 rs3h, rs3l], axis=1)

    @pl.when((g >= NB0) & (g < NB0 + NB2))
    def _phase1():
        rows = pl.ds((g - NB0) * BM2, BM2)
        for c in range(C):
            x = X_s[c]
            cs = jnp.sum(x.astype(jnp.float32), axis=0, keepdims=True)
            full = jnp.dot(e2_s[c, rows, :], x,
                           preferred_element_type=jnp.float32) + cs
            rs = full[:, DO:DO + 1]
            v = (full[:, 2 * DO + 1:2 * DO + 2]
                 + full[:, 2 * DO + 2:2 * DO + 3]) / rs
            vh = v.astype(jnp.bfloat16)
            vl = (v - vh.astype(jnp.float32)).astype(jnp.bfloat16)
            zmain = (full[:, 0:2 * DO + 1] / rs).astype(jnp.bfloat16)
            Z_s[c, rows, :] = jnp.concatenate([zmain, vh, vl], axis=1)

    @pl.when(g >= NB0 + NB2)
    def _phase2():
        rows = pl.ds((g - NB0 - NB2) * BM2, BM2)
        for c in range(C):
            z = Z_s[c]
            e1r = e1_s[c, rows, :]
            cs = jnp.sum(z.astype(jnp.float32), axis=0, keepdims=True)
            full = jnp.dot(e1r, z, preferred_element_type=jnp.float32) + cs
            rs1 = full[:, DO:DO + 1]
            s2 = (full[:, 2 * DO + 1:2 * DO + 2]
                  + full[:, 2 * DO + 2:2 * DO + 3]) / rs1
            s2 = jnp.where(s2 == 0.0, 1.0, s2)
            hwa = hwa_s[c]
            csh = jnp.sum(hwa.astype(jnp.float32), axis=0, keepdims=True)
            full1 = jnp.dot(e1r, hwa, preferred_element_type=jnp.float32) + csh
            o_ref[c, 0] = full1[:, 0:DO] / full1[:, DO:DO + 1] + b_ref[c]
            o_ref[c, 1] = full[:, 0:DO] / rs1 + b_ref[c]
            o_ref[c, 2] = full[:, DO + 1:2 * DO + 1] / (rs1 * s2) + b_ref[c]


def kernel(A, h, W_l0_c1, W_l0_c2, W_l1_c1, gcn_W, gcn_b):
    T, N, _ = A.shape
    C, D, DO = gcn_W.shape
    BM0, BM2 = _BM0, _BM2
    NB0, NB2 = N // BM0, N // BM2
    HWW = DO + 1
    XW = 2 * DO + 3
    b2d = gcn_b.reshape(C, 1, DO)

    return pl.pallas_call(
        functools.partial(_body, C=C, T=T, DO=DO,
                          BM0=BM0, NB0=NB0, BM2=BM2, NB2=NB2),
        grid=(NB0 + 2 * NB2,),
        in_specs=[
            pl.BlockSpec((T, BM0, N),
                         lambda g: (0, jnp.minimum(g, NB0 - 1), 0)),
            pl.BlockSpec(memory_space=pltpu.SMEM),
            pl.BlockSpec(memory_space=pltpu.SMEM),
            pl.BlockSpec(memory_space=pltpu.SMEM),
            pl.BlockSpec((N, D), lambda g: (0, 0)),
            pl.BlockSpec((C, D, DO), lambda g: (0, 0, 0)),
            pl.BlockSpec((C, 1, DO), lambda g: (0, 0, 0)),
        ],
        out_specs=pl.BlockSpec(
            (C, 3, BM2, DO),
            lambda g: (0, 0, jnp.maximum(g - (NB0 + NB2), 0), 0)),
        out_shape=jax.ShapeDtypeStruct((C, 3, N, DO), jnp.float32),
        scratch_shapes=[
            pltpu.VMEM((C, N, N), jnp.bfloat16),
            pltpu.VMEM((C, N, N), jnp.bfloat16),
            pltpu.VMEM((C, N, XW), jnp.bfloat16),
            pltpu.VMEM((C, N, XW), jnp.bfloat16),
            pltpu.VMEM((C, N, HWW), jnp.bfloat16),
        ],
        compiler_params=pltpu.CompilerParams(
            dimension_semantics=("arbitrary",),
            vmem_limit_bytes=64 * 1024 * 1024),
    )(A, W_l0_c1, W_l0_c2, W_l1_c1, h, gcn_W, b2d)


# speedup vs baseline: 8.6503x; 1.0416x over previous
import functools

import jax
import jax.numpy as jnp
from jax.experimental import pallas as pl
from jax.experimental.pallas import tpu as pltpu

_BM0 = 256
_BM1 = 1024
_BM2 = 512


def _expm1_poly(x):
    return x * (1.0 + x * (0.5 + x * (1.0 / 6.0)))


def _body(A_ref, W1, W2, W3, h_ref, w_ref, b_ref, o_ref,
          e1_s, e2_s, X_s, Z_s, hwa_s,
          *, C, T, DO, BM0, NB0, BM1, NB1, BM2, NB2):
    g = pl.program_id(0)

    @pl.when(g == 0)
    def _init():
        n = h_ref.shape[0]
        ones = jnp.ones((n, 1), jnp.float32)
        for c in range(C):
            hw = jnp.dot(h_ref[...], w_ref[c],
                         preferred_element_type=jnp.float32)
            hwa_s[c] = jnp.concatenate([hw, ones], axis=1).astype(jnp.bfloat16)

    @pl.when(g < NB0)
    def _phase0():
        rows = pl.ds(g * BM0, BM0)
        ab = [A_ref[t].astype(jnp.bfloat16) for t in range(T)]
        for c in range(C):
            w1 = [W1[c, t].astype(jnp.bfloat16) for t in range(T)]
            w2 = [W2[c, t].astype(jnp.bfloat16) for t in range(T)]
            b1 = w1[0] * ab[0]
            b2 = w2[0] * ab[0]
            b3 = W3[c, 0] * A_ref[0]
            for t in range(1, T):
                b1 += w1[t] * ab[t]
                b2 += w2[t] * ab[t]
                b3 += W3[c, t] * A_ref[t]
            e1 = _expm1_poly(b1)
            e2 = _expm1_poly(b2)
            e1_s[c, rows, :] = e1
            e2_s[c, rows, :] = e2
            hwa = hwa_s[c]
            b3h = b3.astype(jnp.bfloat16)
            b3l = (b3 - b3h.astype(jnp.float32)).astype(jnp.bfloat16)
            p3 = (jnp.dot(b3h, hwa, preferred_element_type=jnp.float32)
                  + jnp.dot(b3l, hwa, preferred_element_type=jnp.float32))
            rs3 = p3[:, DO:DO + 1]
            rs3h = rs3.astype(jnp.bfloat16)
            rs3l = (rs3 - rs3h.astype(jnp.float32)).astype(jnp.bfloat16)
            X_s[c, rows, :] = jnp.concatenate(
                [hwa_s[c, rows, :], p3[:, 0:DO].astype(jnp.bfloat16),
                 rs3h, rs3l], axis=1)

    @pl.when((g >= NB0) & (g < NB0 + NB1))
    def _phase1():
        rows = pl.ds((g - NB0) * BM1, BM1)
        for c in range(C):
            x = X_s[c]
            cs = jnp.sum(x.astype(jnp.float32), axis=0, keepdims=True)
            full = jnp.dot(e2_s[c, rows, :], x,
                           preferred_element_type=jnp.float32) + cs
            rs = full[:, DO:DO + 1]
            v = (full[:, 2 * DO + 1:2 * DO + 2]
                 + full[:, 2 * DO + 2:2 * DO + 3]) / rs
            vh = v.astype(jnp.bfloat16)
            vl = (v - vh.astype(jnp.float32)).astype(jnp.bfloat16)
            zmain = (full[:, 0:2 * DO + 1] / rs).astype(jnp.bfloat16)
            Z_s[c, rows, :] = jnp.concatenate([zmain, vh, vl], axis=1)

    @pl.when(g >= NB0 + NB1)
    def _phase2():
        rows = pl.ds((g - NB0 - NB1) * BM2, BM2)
        for c in range(C):
            z = Z_s[c]
            e1r = e1_s[c, rows, :]
            cs = jnp.sum(z.astype(jnp.float32), axis=0, keepdims=True)
            full = jnp.dot(e1r, z, preferred_element_type=jnp.float32) + cs
            rs1 = full[:, DO:DO + 1]
            s2 = (full[:, 2 * DO + 1:2 * DO + 2]
                  + full[:, 2 * DO + 2:2 * DO + 3]) / rs1
            s2 = jnp.where(s2 == 0.0, 1.0, s2)
            hwa = hwa_s[c]
            csh = jnp.sum(hwa.astype(jnp.float32), axis=0, keepdims=True)
            full1 = jnp.dot(e1r, hwa, preferred_element_type=jnp.float32) + csh
            o_ref[c, 0] = full1[:, 0:DO] / full1[:, DO:DO + 1] + b_ref[c]
            o_ref[c, 1] = full[:, 0:DO] / rs1 + b_ref[c]
            o_ref[c, 2] = full[:, DO + 1:2 * DO + 1] / (rs1 * s2) + b_ref[c]


def kernel(A, h, W_l0_c1, W_l0_c2, W_l1_c1, gcn_W, gcn_b):
    T, N, _ = A.shape
    C, D, DO = gcn_W.shape
    BM0, BM1, BM2 = _BM0, _BM1, _BM2
    NB0, NB1, NB2 = N // BM0, N // BM1, N // BM2
    HWW = DO + 1
    XW = 2 * DO + 3
    b2d = gcn_b.reshape(C, 1, DO)

    return pl.pallas_call(
        functools.partial(_body, C=C, T=T, DO=DO, BM0=BM0, NB0=NB0,
                          BM1=BM1, NB1=NB1, BM2=BM2, NB2=NB2),
        grid=(NB0 + NB1 + NB2,),
        in_specs=[
            pl.BlockSpec((T, BM0, N),
                         lambda g: (0, jnp.minimum(g, NB0 - 1), 0)),
            pl.BlockSpec(memory_space=pltpu.SMEM),
            pl.BlockSpec(memory_space=pltpu.SMEM),
            pl.BlockSpec(memory_space=pltpu.SMEM),
            pl.BlockSpec((N, D), lambda g: (0, 0)),
            pl.BlockSpec((C, D, DO), lambda g: (0, 0, 0)),
            pl.BlockSpec((C, 1, DO), lambda g: (0, 0, 0)),
        ],
        out_specs=pl.BlockSpec(
            (C, 3, BM2, DO),
            lambda g: (0, 0, jnp.maximum(g - (NB0 + NB1), 0), 0)),
        out_shape=jax.ShapeDtypeStruct((C, 3, N, DO), jnp.float32),
        scratch_shapes=[
            pltpu.VMEM((C, N, N), jnp.bfloat16),
            pltpu.VMEM((C, N, N), jnp.bfloat16),
            pltpu.VMEM((C, N, XW), jnp.bfloat16),
            pltpu.VMEM((C, N, XW), jnp.bfloat16),
            pltpu.VMEM((C, N, HWW), jnp.bfloat16),
        ],
        compiler_params=pltpu.CompilerParams(
            dimension_semantics=("arbitrary",),
            vmem_limit_bytes=64 * 1024 * 1024),
    )(A, W_l0_c1, W_l0_c2, W_l1_c1, h, gcn_W, b2d)


# speedup vs baseline: 8.7121x; 1.0071x over previous
import functools

import jax
import jax.numpy as jnp
from jax.experimental import pallas as pl
from jax.experimental.pallas import tpu as pltpu

_BM0 = 256
_BM1 = 1024
_BM2 = 512
_AW = 8


def _expm1_poly(x):
    return x * (1.0 + x * (0.5 + x * (1.0 / 6.0)))


def _hilo_aux(val_f32, nrows):
    hi = val_f32.astype(jnp.bfloat16)
    lo = (val_f32 - hi.astype(jnp.float32)).astype(jnp.bfloat16)
    ones = jnp.ones((nrows, 1), jnp.bfloat16)
    zeros = jnp.zeros((nrows, _AW - 3), jnp.bfloat16)
    return jnp.concatenate([ones, hi, lo, zeros], axis=1)


def _body(A_ref, W1, W2, W3, h_ref, w_ref, b_ref, o_ref,
          e1_s, e2_s, X_s, Xa_s, Z_s, Za_s, hwa_s,
          *, C, T, DO, BM0, NB0, BM1, NB1, BM2, NB2):
    g = pl.program_id(0)

    @pl.when(g == 0)
    def _init():
        n = h_ref.shape[0]
        ones = jnp.ones((n, 1), jnp.float32)
        for c in range(C):
            hw = jnp.dot(h_ref[...], w_ref[c],
                         preferred_element_type=jnp.float32)
            hwa_s[c] = jnp.concatenate([hw, ones], axis=1).astype(jnp.bfloat16)

    @pl.when(g < NB0)
    def _phase0():
        rows = pl.ds(g * BM0, BM0)
        ab = [A_ref[t].astype(jnp.bfloat16) for t in range(T)]
        for c in range(C):
            w1 = [W1[c, t].astype(jnp.bfloat16) for t in range(T)]
            w2 = [W2[c, t].astype(jnp.bfloat16) for t in range(T)]
            b1 = w1[0] * ab[0]
            b2 = w2[0] * ab[0]
            b3 = W3[c, 0] * A_ref[0]
            for t in range(1, T):
                b1 += w1[t] * ab[t]
                b2 += w2[t] * ab[t]
                b3 += W3[c, t] * A_ref[t]
            e1 = _expm1_poly(b1)
            e2 = _expm1_poly(b2)
            e1_s[c, rows, :] = e1
            e2_s[c, rows, :] = e2
            hwa = hwa_s[c]
            b3h = b3.astype(jnp.bfloat16)
            b3l = (b3 - b3h.astype(jnp.float32)).astype(jnp.bfloat16)
            p3 = (jnp.dot(b3h, hwa, preferred_element_type=jnp.float32)
                  + jnp.dot(b3l, hwa, preferred_element_type=jnp.float32))
            X_s[c, rows, :] = jnp.concatenate(
                [hwa_s[c, rows, 0:DO], p3[:, 0:DO].astype(jnp.bfloat16)],
                axis=1)
            Xa_s[c, rows, :] = _hilo_aux(p3[:, DO:DO + 1], BM0)

    @pl.when((g >= NB0) & (g < NB0 + NB1))
    def _phase1():
        rows = pl.ds((g - NB0) * BM1, BM1)
        for c in range(C):
            x = X_s[c]
            xa = Xa_s[c]
            cs = jnp.sum(x.astype(jnp.float32), axis=0, keepdims=True)
            csa = jnp.sum(xa.astype(jnp.float32), axis=0, keepdims=True)
            e2r = e2_s[c, rows, :]
            full = jnp.dot(e2r, x, preferred_element_type=jnp.float32) + cs
            fa = jnp.dot(e2r, xa, preferred_element_type=jnp.float32) + csa
            rs = fa[:, 0:1]
            v = (fa[:, 1:2] + fa[:, 2:3]) / rs
            Z_s[c, rows, :] = (full / rs).astype(jnp.bfloat16)
            Za_s[c, rows, :] = _hilo_aux(v, BM1)

    @pl.when(g >= NB0 + NB1)
    def _phase2():
        rows = pl.ds((g - NB0 - NB1) * BM2, BM2)
        for c in range(C):
            z = Z_s[c]
            za = Za_s[c]
            e1r = e1_s[c, rows, :]
            cs = jnp.sum(z.astype(jnp.float32), axis=0, keepdims=True)
            csa = jnp.sum(za.astype(jnp.float32), axis=0, keepdims=True)
            full = jnp.dot(e1r, z, preferred_element_type=jnp.float32) + cs
            fa = jnp.dot(e1r, za, preferred_element_type=jnp.float32) + csa
            rs1 = fa[:, 0:1]
            s2 = (fa[:, 1:2] + fa[:, 2:3]) / rs1
            s2 = jnp.where(s2 == 0.0, 1.0, s2)
            hw = hwa_s[c][:, 0:DO]
            csh = jnp.sum(hw.astype(jnp.float32), axis=0, keepdims=True)
            full1 = jnp.dot(e1r, hw, preferred_element_type=jnp.float32) + csh
            o_ref[c, 0] = full1 / rs1 + b_ref[c]
            o_ref[c, 1] = full[:, 0:DO] / rs1 + b_ref[c]
            o_ref[c, 2] = full[:, DO:2 * DO] / (rs1 * s2) + b_ref[c]


def kernel(A, h, W_l0_c1, W_l0_c2, W_l1_c1, gcn_W, gcn_b):
    T, N, _ = A.shape
    C, D, DO = gcn_W.shape
    BM0, BM1, BM2 = _BM0, _BM1, _BM2
    NB0, NB1, NB2 = N // BM0, N // BM1, N // BM2
    HWW = DO + 1
    XW = 2 * DO
    b2d = gcn_b.reshape(C, 1, DO)

    return pl.pallas_call(
        functools.partial(_body, C=C, T=T, DO=DO, BM0=BM0, NB0=NB0,
                          BM1=BM1, NB1=NB1, BM2=BM2, NB2=NB2),
        grid=(NB0 + NB1 + NB2,),
        in_specs=[
            pl.BlockSpec((T, BM0, N),
                         lambda g: (0, jnp.minimum(g, NB0 - 1), 0)),
            pl.BlockSpec(memory_space=pltpu.SMEM),
            pl.BlockSpec(memory_space=pltpu.SMEM),
            pl.BlockSpec(memory_space=pltpu.SMEM),
            pl.BlockSpec((N, D), lambda g: (0, 0)),
            pl.BlockSpec((C, D, DO), lambda g: (0, 0, 0)),
            pl.BlockSpec((C, 1, DO), lambda g: (0, 0, 0)),
        ],
        out_specs=pl.BlockSpec(
            (C, 3, BM2, DO),
            lambda g: (0, 0, jnp.maximum(g - (NB0 + NB1), 0), 0)),
        out_shape=jax.ShapeDtypeStruct((C, 3, N, DO), jnp.float32),
        scratch_shapes=[
            pltpu.VMEM((C, N, N), jnp.bfloat16),
            pltpu.VMEM((C, N, N), jnp.bfloat16),
            pltpu.VMEM((C, N, XW), jnp.bfloat16),
            pltpu.VMEM((C, N, _AW), jnp.bfloat16),
            pltpu.VMEM((C, N, XW), jnp.bfloat16),
            pltpu.VMEM((C, N, _AW), jnp.bfloat16),
            pltpu.VMEM((C, N, HWW), jnp.bfloat16),
        ],
        compiler_params=pltpu.CompilerParams(
            dimension_semantics=("arbitrary",),
            vmem_limit_bytes=64 * 1024 * 1024),
    )(A, W_l0_c1, W_l0_c2, W_l1_c1, h, gcn_W, b2d)


# speedup vs baseline: 8.7354x; 1.0027x over previous
import functools

import jax
import jax.numpy as jnp
from jax.experimental import pallas as pl
from jax.experimental.pallas import tpu as pltpu

_BM0 = 256
_BM1 = 2048
_BM2 = 512
_AW = 8


def _expm1_poly(x):
    return x * (1.0 + x * (0.5 + x * (1.0 / 6.0)))


def _hilo_aux(val_f32, nrows):
    hi = val_f32.astype(jnp.bfloat16)
    lo = (val_f32 - hi.astype(jnp.float32)).astype(jnp.bfloat16)
    ones = jnp.ones((nrows, 1), jnp.bfloat16)
    zeros = jnp.zeros((nrows, _AW - 3), jnp.bfloat16)
    return jnp.concatenate([ones, hi, lo, zeros], axis=1)


def _body(A_ref, W1, W2, W3, h_ref, w_ref, b_ref, o_ref,
          e1_s, e2_s, X_s, Xa_s, Z_s, Za_s, hwa_s,
          *, C, T, DO, BM0, NB0, BM1, NB1, BM2, NB2):
    g = pl.program_id(0)

    @pl.when(g == 0)
    def _init():
        n = h_ref.shape[0]
        ones = jnp.ones((n, 1), jnp.float32)
        for c in range(C):
            hw = jnp.dot(h_ref[...], w_ref[c],
                         preferred_element_type=jnp.float32)
            hwa_s[c] = jnp.concatenate([hw, ones], axis=1).astype(jnp.bfloat16)

    @pl.when(g < NB0)
    def _phase0():
        rows = pl.ds(g * BM0, BM0)
        ab = [A_ref[t].astype(jnp.bfloat16) for t in range(T)]
        for c in range(C):
            w1 = [W1[c, t].astype(jnp.bfloat16) for t in range(T)]
            w2 = [W2[c, t].astype(jnp.bfloat16) for t in range(T)]
            b1 = w1[0] * ab[0]
            b2 = w2[0] * ab[0]
            b3 = W3[c, 0] * A_ref[0]
            for t in range(1, T):
                b1 += w1[t] * ab[t]
                b2 += w2[t] * ab[t]
                b3 += W3[c, t] * A_ref[t]
            e1 = _expm1_poly(b1)
            e2 = _expm1_poly(b2)
            e1_s[c, rows, :] = e1
            e2_s[c, rows, :] = e2
            hwa = hwa_s[c]
            b3h = b3.astype(jnp.bfloat16)
            b3l = (b3 - b3h.astype(jnp.float32)).astype(jnp.bfloat16)
            p3 = (jnp.dot(b3h, hwa, preferred_element_type=jnp.float32)
                  + jnp.dot(b3l, hwa, preferred_element_type=jnp.float32))
            X_s[c, rows, :] = jnp.concatenate(
                [hwa_s[c, rows, 0:DO], p3[:, 0:DO].astype(jnp.bfloat16)],
                axis=1)
            Xa_s[c, rows, :] = _hilo_aux(p3[:, DO:DO + 1], BM0)

    @pl.when((g >= NB0) & (g < NB0 + NB1))
    def _phase1():
        rows = pl.ds((g - NB0) * BM1, BM1)
        for c in range(C):
            x = X_s[c]
            xa = Xa_s[c]
            cs = jnp.sum(x.astype(jnp.float32), axis=0, keepdims=True)
            csa = jnp.sum(xa.astype(jnp.float32), axis=0, keepdims=True)
            e2r = e2_s[c, rows, :]
            full = jnp.dot(e2r, x, preferred_element_type=jnp.float32) + cs
            fa = jnp.dot(e2r, xa, preferred_element_type=jnp.float32) + csa
            rs = fa[:, 0:1]
            v = (fa[:, 1:2] + fa[:, 2:3]) / rs
            Z_s[c, rows, :] = (full / rs).astype(jnp.bfloat16)
            Za_s[c, rows, :] = _hilo_aux(v, BM1)

    @pl.when(g >= NB0 + NB1)
    def _phase2():
        rows = pl.ds((g - NB0 - NB1) * BM2, BM2)
        for c in range(C):
            z = Z_s[c]
            za = Za_s[c]
            e1r = e1_s[c, rows, :]
            cs = jnp.sum(z.astype(jnp.float32), axis=0, keepdims=True)
            csa = jnp.sum(za.astype(jnp.float32), axis=0, keepdims=True)
            full = jnp.dot(e1r, z, preferred_element_type=jnp.float32) + cs
            fa = jnp.dot(e1r, za, preferred_element_type=jnp.float32) + csa
            rs1 = fa[:, 0:1]
            s2 = (fa[:, 1:2] + fa[:, 2:3]) / rs1
            s2 = jnp.where(s2 == 0.0, 1.0, s2)
            hw = hwa_s[c][:, 0:DO]
            csh = jnp.sum(hw.astype(jnp.float32), axis=0, keepdims=True)
            full1 = jnp.dot(e1r, hw, preferred_element_type=jnp.float32) + csh
            o_ref[c, 0] = full1 / rs1 + b_ref[c]
            o_ref[c, 1] = full[:, 0:DO] / rs1 + b_ref[c]
            o_ref[c, 2] = full[:, DO:2 * DO] / (rs1 * s2) + b_ref[c]


def kernel(A, h, W_l0_c1, W_l0_c2, W_l1_c1, gcn_W, gcn_b):
    T, N, _ = A.shape
    C, D, DO = gcn_W.shape
    BM0, BM1, BM2 = _BM0, _BM1, _BM2
    NB0, NB1, NB2 = N // BM0, N // BM1, N // BM2
    HWW = DO + 1
    XW = 2 * DO
    b2d = gcn_b.reshape(C, 1, DO)

    return pl.pallas_call(
        functools.partial(_body, C=C, T=T, DO=DO, BM0=BM0, NB0=NB0,
                          BM1=BM1, NB1=NB1, BM2=BM2, NB2=NB2),
        grid=(NB0 + NB1 + NB2,),
        in_specs=[
            pl.BlockSpec((T, BM0, N),
                         lambda g: (0, jnp.minimum(g, NB0 - 1), 0)),
            pl.BlockSpec(memory_space=pltpu.SMEM),
            pl.BlockSpec(memory_space=pltpu.SMEM),
            pl.BlockSpec(memory_space=pltpu.SMEM),
            pl.BlockSpec((N, D), lambda g: (0, 0)),
            pl.BlockSpec((C, D, DO), lambda g: (0, 0, 0)),
            pl.BlockSpec((C, 1, DO), lambda g: (0, 0, 0)),
        ],
        out_specs=pl.BlockSpec(
            (C, 3, BM2, DO),
            lambda g: (0, 0, jnp.maximum(g - (NB0 + NB1), 0), 0)),
        out_shape=jax.ShapeDtypeStruct((C, 3, N, DO), jnp.float32),
        scratch_shapes=[
            pltpu.VMEM((C, N, N), jnp.bfloat16),
            pltpu.VMEM((C, N, N), jnp.bfloat16),
            pltpu.VMEM((C, N, XW), jnp.bfloat16),
            pltpu.VMEM((C, N, _AW), jnp.bfloat16),
            pltpu.VMEM((C, N, XW), jnp.bfloat16),
            pltpu.VMEM((C, N, _AW), jnp.bfloat16),
            pltpu.VMEM((C, N, HWW), jnp.bfloat16),
        ],
        compiler_params=pltpu.CompilerParams(
            dimension_semantics=("arbitrary",),
            vmem_limit_bytes=64 * 1024 * 1024),
    )(A, W_l0_c1, W_l0_c2, W_l1_c1, h, gcn_W, b2d)
